# baseline (speedup 1.0000x reference)
import functools

import jax
import jax.numpy as jnp
from jax import lax
from jax.experimental import pallas as pl
from jax.experimental.pallas import tpu as pltpu
from jax.experimental.pallas import tpu_sc as plsc

_NUM_EXPERTS = 64
_TOP_K = 8
_ALPHA = 0.01
_GAMMA = 0.001
_ROWS = 32768
_BLOCK = 2048

_NC = 2
_NS = 16
_NW = _NC * _NS
_L = 16
_RPW = _ROWS // _NW
_G = 4
_SB = _G * _L
_NSB = _RPW // _SB


def _sc_body(x_hbm, out_hbm, xbuf, cntbuf, sem):
    c = lax.axis_index("c")
    s = lax.axis_index("s")
    wid = s * _NC + c
    row0 = wid * _RPW

    pltpu.async_copy(x_hbm.at[pl.ds(row0 * _NUM_EXPERTS, _RPW * _NUM_EXPERTS)],
                     xbuf, sem).wait()

    iota = lax.broadcasted_iota(jnp.int32, (_L,), 0)
    rowstride = iota * _NUM_EXPERTS
    neginf = jnp.full((_L,), -jnp.inf, jnp.float32)
    zeros = jnp.zeros((_L,), jnp.float32)

    def superblock(sb, cacc):
        base = [(sb * _SB + g * _L) * _NUM_EXPERTS + rowstride
                for g in range(_G)]

        def step(e, r):
            r = list(r)
            for g in range(_G):
                t = plsc.load_gather(xbuf, [base[g] + e])
                for i in range(_TOP_K):
                    hi = jnp.maximum(r[g * _TOP_K + i], t)
                    t = jnp.minimum(r[g * _TOP_K + i], t)
                    r[g * _TOP_K + i] = hi
            return tuple(r)

        rfin = lax.fori_loop(0, _NUM_EXPERTS, step,
                             tuple([neginf] * (_G * _TOP_K)))
        thr = [rfin[g * _TOP_K + _TOP_K - 1] for g in range(_G)]

        cacc = list(cacc)
        for e in range(_NUM_EXPERTS):
            pc = None
            for g in range(_G):
                v = plsc.load_gather(xbuf, [base[g] + e])
                p = plsc.all_reduce_population_count(v >= thr[g])
                pc = p if pc is None else pc + p
            onehot = (iota == (e % _L)).astype(jnp.float32)
            j = e // _L
            cacc[j] = cacc[j] + onehot * pc.astype(jnp.float32)
        return tuple(cacc)

    cacc = lax.fori_loop(0, _NSB, superblock, tuple([zeros] * 4))
    for j in range(4):
        cntbuf[pl.ds(j * _L, _L)] = cacc[j]
    pltpu.sync_copy(cntbuf, out_hbm.at[wid])


@functools.partial(
    pl.kernel,
    out_type=jax.ShapeDtypeStruct((_NW, _NUM_EXPERTS), jnp.float32),
    mesh=plsc.VectorSubcoreMesh(core_axis_name="c", subcore_axis_name="s"),
    scratch_types=[
        pltpu.VMEM((_RPW * _NUM_EXPERTS,), jnp.float32),
        pltpu.VMEM((_NUM_EXPERTS,), jnp.float32),
        pltpu.SemaphoreType.DMA,
    ],
    compiler_params=pltpu.CompilerParams(needs_layout_passes=False),
)
def _sc_counts(x_hbm, out_hbm, xbuf, cntbuf, sem):
    _sc_body(x_hbm, out_hbm, xbuf, cntbuf, sem)


def _tc_body(x_ref, acc_ref):
    pi = pl.program_id(0)

    @pl.when(pi == 0)
    def _init():
        acc_ref[...] = jnp.zeros_like(acc_ref)

    x = x_ref[...]
    m = jnp.max(x, axis=1, keepdims=True)
    ex = jnp.exp(x - m)
    s = jnp.sum(ex, axis=1, keepdims=True)
    lse = m + jnp.log(s)
    z_part = jnp.sum(lse * lse, keepdims=True)
    probs = ex / s
    prob_part = jnp.sum(probs, axis=0, keepdims=True)

    acc_ref[0:1, 0:_NUM_EXPERTS] += prob_part
    acc_ref[1:2, 0:1] += z_part


def _tc_softmax_stats(router_logits):
    return pl.pallas_call(
        _tc_body,
        grid=(_ROWS // _BLOCK,),
        in_specs=[pl.BlockSpec((_BLOCK, _NUM_EXPERTS), lambda i: (i, 0))],
        out_specs=pl.BlockSpec((8, 128), lambda i: (0, 0)),
        out_shape=jax.ShapeDtypeStruct((8, 128), jnp.float32),
    )(router_logits)


@jax.jit
def kernel(router_logits):
    cnt_parts = _sc_counts(router_logits.reshape(-1))
    acc = _tc_softmax_stats(router_logits)
    inv_n = 1.0 / _ROWS
    expert_prob = acc[0, :_NUM_EXPERTS] * inv_n
    expert_freq = jnp.sum(cnt_parts, axis=0) * inv_n
    z_loss = acc[1, 0] * inv_n
    global_loss = _NUM_EXPERTS * jnp.sum(expert_prob * expert_freq)
    return _ALPHA * global_loss + _GAMMA * z_loss


# speedup vs baseline: 1.2159x; 1.2159x over previous
import functools

import jax
import jax.numpy as jnp
from jax import lax
from jax.experimental import pallas as pl
from jax.experimental.pallas import tpu as pltpu
from jax.experimental.pallas import tpu_sc as plsc

_NUM_EXPERTS = 64
_TOP_K = 8
_ALPHA = 0.01
_GAMMA = 0.001
_ROWS = 32768
_BLOCK = 2048

_NC = 2
_NS = 16
_NW = _NC * _NS
_L = 16
_RPW = _ROWS // _NW
_G = 2
_SB = _G * _L
_NSB = _RPW // _SB

_SORT8 = [(0, 1), (2, 3), (4, 5), (6, 7),
          (0, 2), (1, 3), (4, 6), (5, 7),
          (1, 2), (5, 6),
          (0, 4), (1, 5), (2, 6), (3, 7),
          (2, 4), (3, 5),
          (1, 2), (3, 4), (5, 6)]
_BITONIC8 = [(0, 4), (1, 5), (2, 6), (3, 7),
             (0, 2), (1, 3), (4, 6), (5, 7),
             (0, 1), (2, 3), (4, 5), (6, 7)]


def _sc_body(x_hbm, out_hbm, xbuf, cntbuf, sem):
    c = lax.axis_index("c")
    s = lax.axis_index("s")
    wid = s * _NC + c
    row0 = wid * _RPW

    pltpu.async_copy(x_hbm.at[pl.ds(row0 * _NUM_EXPERTS, _RPW * _NUM_EXPERTS)],
                     xbuf, sem).wait()

    iota = lax.broadcasted_iota(jnp.int32, (_L,), 0)
    rowstride = iota * _NUM_EXPERTS
    neginf = jnp.full((_L,), -jnp.inf, jnp.float32)
    zeros = jnp.zeros((_L,), jnp.float32)

    def superblock(sb, cacc):
        base = [(sb * _SB + g * _L) * _NUM_EXPERTS + rowstride
                for g in range(_G)]

        def step(it, r):
            r = list(r)
            eb = it * _TOP_K
            for g in range(_G):
                ebase = base[g] + eb
                b = [plsc.load_gather(xbuf, [ebase + k])
                     for k in range(_TOP_K)]
                for i, j in _SORT8:
                    lo = jnp.minimum(b[i], b[j])
                    b[j] = jnp.maximum(b[i], b[j])
                    b[i] = lo
                m = [jnp.maximum(r[g * _TOP_K + i], b[i])
                     for i in range(_TOP_K)]
                for i, j in _BITONIC8:
                    hi = jnp.maximum(m[i], m[j])
                    m[j] = jnp.minimum(m[i], m[j])
                    m[i] = hi
                for i in range(_TOP_K):
                    r[g * _TOP_K + i] = m[i]
            return tuple(r)

        rfin = lax.fori_loop(0, _NUM_EXPERTS // _TOP_K, step,
                             tuple([neginf] * (_G * _TOP_K)))
        thr = [rfin[g * _TOP_K + _TOP_K - 1] for g in range(_G)]

        cacc = list(cacc)
        for e in range(_NUM_EXPERTS):
            pc = None
            for g in range(_G):
                v = plsc.load_gather(xbuf, [base[g] + e])
                p = plsc.all_reduce_population_count(v >= thr[g])
                pc = p if pc is None else pc + p
            onehot = (iota == (e % _L)).astype(jnp.float32)
            j = e // _L
            cacc[j] = cacc[j] + onehot * pc.astype(jnp.float32)
        return tuple(cacc)

    cacc = lax.fori_loop(0, _NSB, superblock, tuple([zeros] * 4))
    for j in range(4):
        cntbuf[pl.ds(j * _L, _L)] = cacc[j]
    pltpu.sync_copy(cntbuf, out_hbm.at[wid])


@functools.partial(
    pl.kernel,
    out_type=jax.ShapeDtypeStruct((_NW, _NUM_EXPERTS), jnp.float32),
    mesh=plsc.VectorSubcoreMesh(core_axis_name="c", subcore_axis_name="s"),
    scratch_types=[
        pltpu.VMEM((_RPW * _NUM_EXPERTS,), jnp.float32),
        pltpu.VMEM((_NUM_EXPERTS,), jnp.float32),
        pltpu.SemaphoreType.DMA,
    ],
    compiler_params=pltpu.CompilerParams(needs_layout_passes=False),
)
def _sc_counts(x_hbm, out_hbm, xbuf, cntbuf, sem):
    _sc_body(x_hbm, out_hbm, xbuf, cntbuf, sem)


def _tc_body(x_ref, acc_ref):
    pi = pl.program_id(0)

    @pl.when(pi == 0)
    def _init():
        acc_ref[...] = jnp.zeros_like(acc_ref)

    x = x_ref[...]
    m = jnp.max(x, axis=1, keepdims=True)
    ex = jnp.exp(x - m)
    s = jnp.sum(ex, axis=1, keepdims=True)
    lse = m + jnp.log(s)
    z_part = jnp.sum(lse * lse, keepdims=True)
    probs = ex / s
    prob_part = jnp.sum(probs, axis=0, keepdims=True)

    acc_ref[0:1, 0:_NUM_EXPERTS] += prob_part
    acc_ref[1:2, 0:1] += z_part


def _tc_softmax_stats(router_logits):
    return pl.pallas_call(
        _tc_body,
        grid=(_ROWS // _BLOCK,),
        in_specs=[pl.BlockSpec((_BLOCK, _NUM_EXPERTS), lambda i: (i, 0))],
        out_specs=pl.BlockSpec((8, 128), lambda i: (0, 0)),
        out_shape=jax.ShapeDtypeStruct((8, 128), jnp.float32),
    )(router_logits)


@jax.jit
def kernel(router_logits):
    cnt_parts = _sc_counts(router_logits.reshape(-1))
    acc = _tc_softmax_stats(router_logits)
    inv_n = 1.0 / _ROWS
    expert_prob = acc[0, :_NUM_EXPERTS] * inv_n
    expert_freq = jnp.sum(cnt_parts, axis=0) * inv_n
    z_loss = acc[1, 0] * inv_n
    global_loss = _NUM_EXPERTS * jnp.sum(expert_prob * expert_freq)
    return _ALPHA * global_loss + _GAMMA * z_loss


# speedup vs baseline: 1.4813x; 1.2183x over previous
import functools

import jax
import jax.numpy as jnp
from jax import lax
from jax.experimental import pallas as pl
from jax.experimental.pallas import tpu as pltpu
from jax.experimental.pallas import tpu_sc as plsc

_NUM_EXPERTS = 64
_TOP_K = 8
_ALPHA = 0.01
_GAMMA = 0.001
_ROWS = 32768
_BLOCK = 2048

_NC = 2
_NS = 16
_NW = _NC * _NS
_L = 16
_RPW = _ROWS // _NW
_G = 2
_SB = _G * _L
_NSB = _RPW // _SB

_SORT8 = [(0, 1), (2, 3), (4, 5), (6, 7),
          (0, 2), (1, 3), (4, 6), (5, 7),
          (1, 2), (5, 6),
          (0, 4), (1, 5), (2, 6), (3, 7),
          (2, 4), (3, 5),
          (1, 2), (3, 4), (5, 6)]
_BITONIC8 = [(0, 4), (1, 5), (2, 6), (3, 7),
             (0, 2), (1, 3), (4, 6), (5, 7),
             (0, 1), (2, 3), (4, 5), (6, 7)]


def _sc_body(x_hbm, out_hbm, xbuf, cntbuf, sem):
    c = lax.axis_index("c")
    s = lax.axis_index("s")
    wid = s * _NC + c
    row0 = wid * _RPW

    pltpu.async_copy(x_hbm.at[pl.ds(row0 * _NUM_EXPERTS, _RPW * _NUM_EXPERTS)],
                     xbuf, sem).wait()

    iota = lax.broadcasted_iota(jnp.int32, (_L,), 0)
    rowstride = iota * _NUM_EXPERTS
    neginf = jnp.full((_L,), -jnp.inf, jnp.float32)
    zeros = jnp.zeros((_L,), jnp.float32)

    def superblock(sb, cacc):
        base = [(sb * _SB + g * _L) * _NUM_EXPERTS + rowstride
                for g in range(_G)]

        r = [neginf] * (_G * _TOP_K)
        for it in range(_NUM_EXPERTS // _TOP_K):
            eb = it * _TOP_K
            for g in range(_G):
                ebase = base[g] + eb
                b = [plsc.load_gather(xbuf, [ebase + k])
                     for k in range(_TOP_K)]
                for i, j in _SORT8:
                    lo = jnp.minimum(b[i], b[j])
                    b[j] = jnp.maximum(b[i], b[j])
                    b[i] = lo
                m = [jnp.maximum(r[g * _TOP_K + i], b[i])
                     for i in range(_TOP_K)]
                for i, j in _BITONIC8:
                    hi = jnp.maximum(m[i], m[j])
                    m[j] = jnp.minimum(m[i], m[j])
                    m[i] = hi
                for i in range(_TOP_K):
                    r[g * _TOP_K + i] = m[i]
        thr = [r[g * _TOP_K + _TOP_K - 1] for g in range(_G)]

        cacc = list(cacc)
        for e in range(_NUM_EXPERTS):
            pc = None
            for g in range(_G):
                v = plsc.load_gather(xbuf, [base[g] + e])
                p = plsc.all_reduce_population_count(v >= thr[g])
                pc = p if pc is None else pc + p
            onehot = (iota == (e % _L)).astype(jnp.float32)
            j = e // _L
            cacc[j] = cacc[j] + onehot * pc.astype(jnp.float32)
        return tuple(cacc)

    cacc = lax.fori_loop(0, _NSB, superblock, tuple([zeros] * 4))
    for j in range(4):
        cntbuf[pl.ds(j * _L, _L)] = cacc[j]
    pltpu.sync_copy(cntbuf, out_hbm.at[wid])


@functools.partial(
    pl.kernel,
    out_type=jax.ShapeDtypeStruct((_NW, _NUM_EXPERTS), jnp.float32),
    mesh=plsc.VectorSubcoreMesh(core_axis_name="c", subcore_axis_name="s"),
    scratch_types=[
        pltpu.VMEM((_RPW * _NUM_EXPERTS,), jnp.float32),
        pltpu.VMEM((_NUM_EXPERTS,), jnp.float32),
        pltpu.SemaphoreType.DMA,
    ],
    compiler_params=pltpu.CompilerParams(needs_layout_passes=False),
)
def _sc_counts(x_hbm, out_hbm, xbuf, cntbuf, sem):
    _sc_body(x_hbm, out_hbm, xbuf, cntbuf, sem)


def _tc_body(x_ref, acc_ref):
    pi = pl.program_id(0)

    @pl.when(pi == 0)
    def _init():
        acc_ref[...] = jnp.zeros_like(acc_ref)

    x = x_ref[...]
    m = jnp.max(x, axis=1, keepdims=True)
    ex = jnp.exp(x - m)
    s = jnp.sum(ex, axis=1, keepdims=True)
    lse = m + jnp.log(s)
    z_part = jnp.sum(lse * lse, keepdims=True)
    probs = ex / s
    prob_part = jnp.sum(probs, axis=0, keepdims=True)

    acc_ref[0:1, 0:_NUM_EXPERTS] += prob_part
    acc_ref[1:2, 0:1] += z_part


def _tc_softmax_stats(router_logits):
    return pl.pallas_call(
        _tc_body,
        grid=(_ROWS // _BLOCK,),
        in_specs=[pl.BlockSpec((_BLOCK, _NUM_EXPERTS), lambda i: (i, 0))],
        out_specs=pl.BlockSpec((8, 128), lambda i: (0, 0)),
        out_shape=jax.ShapeDtypeStruct((8, 128), jnp.float32),
    )(router_logits)


@jax.jit
def kernel(router_logits):
    cnt_parts = _sc_counts(router_logits.reshape(-1))
    acc = _tc_softmax_stats(router_logits)
    inv_n = 1.0 / _ROWS
    expert_prob = acc[0, :_NUM_EXPERTS] * inv_n
    expert_freq = jnp.sum(cnt_parts, axis=0) * inv_n
    z_loss = acc[1, 0] * inv_n
    global_loss = _NUM_EXPERTS * jnp.sum(expert_prob * expert_freq)
    return _ALPHA * global_loss + _GAMMA * z_loss


# speedup vs baseline: 1.5713x; 1.0608x over previous
import functools

import jax
import jax.numpy as jnp
from jax import lax
from jax.experimental import pallas as pl
from jax.experimental.pallas import tpu as pltpu
from jax.experimental.pallas import tpu_sc as plsc

_NUM_EXPERTS = 64
_TOP_K = 8
_ALPHA = 0.01
_GAMMA = 0.001
_ROWS = 32768
_BLOCK = 2048

_NC = 2
_NS = 16
_NW = _NC * _NS
_L = 16
_RPW = _ROWS // _NW
_G = 2
_SB = _G * _L
_NSB = _RPW // _SB

_SORT8 = [(0, 1), (2, 3), (4, 5), (6, 7),
          (0, 2), (1, 3), (4, 6), (5, 7),
          (1, 2), (5, 6),
          (0, 4), (1, 5), (2, 6), (3, 7),
          (2, 4), (3, 5),
          (1, 2), (3, 4), (5, 6)]
_BITONIC8 = [(0, 4), (1, 5), (2, 6), (3, 7),
             (0, 2), (1, 3), (4, 6), (5, 7),
             (0, 1), (2, 3), (4, 5), (6, 7)]


def _sc_body(x_hbm, out_hbm, xbuf, cntbuf, thrbuf, sem):
    c = lax.axis_index("c")
    s = lax.axis_index("s")
    wid = s * _NC + c
    row0 = wid * _RPW

    pltpu.async_copy(x_hbm.at[pl.ds(row0 * _NUM_EXPERTS, _RPW * _NUM_EXPERTS)],
                     xbuf, sem).wait()

    iota = lax.broadcasted_iota(jnp.int32, (_L,), 0)
    rowstride = iota * _NUM_EXPERTS
    neginf = jnp.full((_L,), -jnp.inf, jnp.float32)
    zeros = jnp.zeros((_L,), jnp.float32)

    def superblock(sb, cacc):
        base = [(sb * _SB + g * _L) * _NUM_EXPERTS + rowstride
                for g in range(_G)]

        r = [neginf] * (_G * _TOP_K)
        for it in range(_NUM_EXPERTS // _TOP_K):
            eb = it * _TOP_K
            for g in range(_G):
                ebase = base[g] + eb
                b = [plsc.load_gather(xbuf, [ebase + k])
                     for k in range(_TOP_K)]
                for i, j in _SORT8:
                    lo = jnp.minimum(b[i], b[j])
                    b[j] = jnp.maximum(b[i], b[j])
                    b[i] = lo
                m = [jnp.maximum(r[g * _TOP_K + i], b[i])
                     for i in range(_TOP_K)]
                for i, j in _BITONIC8:
                    hi = jnp.maximum(m[i], m[j])
                    m[j] = jnp.minimum(m[i], m[j])
                    m[i] = hi
                for i in range(_TOP_K):
                    r[g * _TOP_K + i] = m[i]
        thr = [r[g * _TOP_K + _TOP_K - 1] for g in range(_G)]

        for g in range(_G):
            thrbuf[pl.ds(g * _L, _L)] = thr[g]
        cacc = list(cacc)
        for g in range(_G):
            for rr in range(_L):
                t16 = plsc.load_gather(
                    thrbuf, [jnp.full((_L,), g * _L + rr, jnp.int32)])
                rowbase = (sb * _SB + g * _L + rr) * _NUM_EXPERTS
                for j in range(_NUM_EXPERTS // _L):
                    v = xbuf[pl.ds(rowbase + j * _L, _L)]
                    cacc[j] = cacc[j] + jnp.where(v >= t16, 1.0, 0.0)
        return tuple(cacc)

    cacc = lax.fori_loop(0, _NSB, superblock, tuple([zeros] * 4))
    for j in range(4):
        cntbuf[pl.ds(j * _L, _L)] = cacc[j]
    pltpu.sync_copy(cntbuf, out_hbm.at[wid])


@functools.partial(
    pl.kernel,
    out_type=jax.ShapeDtypeStruct((_NW, _NUM_EXPERTS), jnp.float32),
    mesh=plsc.VectorSubcoreMesh(core_axis_name="c", subcore_axis_name="s"),
    scratch_types=[
        pltpu.VMEM((_RPW * _NUM_EXPERTS,), jnp.float32),
        pltpu.VMEM((_NUM_EXPERTS,), jnp.float32),
        pltpu.VMEM((_SB,), jnp.float32),
        pltpu.SemaphoreType.DMA,
    ],
    compiler_params=pltpu.CompilerParams(needs_layout_passes=False),
)
def _sc_counts(x_hbm, out_hbm, xbuf, cntbuf, thrbuf, sem):
    _sc_body(x_hbm, out_hbm, xbuf, cntbuf, thrbuf, sem)


def _tc_body(x_ref, acc_ref):
    pi = pl.program_id(0)

    @pl.when(pi == 0)
    def _init():
        acc_ref[...] = jnp.zeros_like(acc_ref)

    x = x_ref[...]
    m = jnp.max(x, axis=1, keepdims=True)
    ex = jnp.exp(x - m)
    s = jnp.sum(ex, axis=1, keepdims=True)
    lse = m + jnp.log(s)
    z_part = jnp.sum(lse * lse, keepdims=True)
    probs = ex / s
    prob_part = jnp.sum(probs, axis=0, keepdims=True)

    acc_ref[0:1, 0:_NUM_EXPERTS] += prob_part
    acc_ref[1:2, 0:1] += z_part


def _tc_softmax_stats(router_logits):
    return pl.pallas_call(
        _tc_body,
        grid=(_ROWS // _BLOCK,),
        in_specs=[pl.BlockSpec((_BLOCK, _NUM_EXPERTS), lambda i: (i, 0))],
        out_specs=pl.BlockSpec((8, 128), lambda i: (0, 0)),
        out_shape=jax.ShapeDtypeStruct((8, 128), jnp.float32),
    )(router_logits)


@jax.jit
def kernel(router_logits):
    cnt_parts = _sc_counts(router_logits.reshape(-1))
    acc = _tc_softmax_stats(router_logits)
    inv_n = 1.0 / _ROWS
    expert_prob = acc[0, :_NUM_EXPERTS] * inv_n
    expert_freq = jnp.sum(cnt_parts, axis=0) * inv_n
    z_loss = acc[1, 0] * inv_n
    global_loss = _NUM_EXPERTS * jnp.sum(expert_prob * expert_freq)
    return _ALPHA * global_loss + _GAMMA * z_loss


# speedup vs baseline: 1.5732x; 1.0012x over previous
import functools

import jax
import jax.numpy as jnp
from jax import lax
from jax.experimental import pallas as pl
from jax.experimental.pallas import tpu as pltpu
from jax.experimental.pallas import tpu_sc as plsc

_NUM_EXPERTS = 64
_TOP_K = 8
_ALPHA = 0.01
_GAMMA = 0.001
_ROWS = 32768
_BLOCK = 2048

_NC = 2
_NS = 16
_NW = _NC * _NS
_L = 16
_RPW = _ROWS // _NW
_G = 2
_SB = _G * _L
_NSB = _RPW // _SB

_SORT8 = [(0, 1), (2, 3), (4, 5), (6, 7),
          (0, 2), (1, 3), (4, 6), (5, 7),
          (1, 2), (5, 6),
          (0, 4), (1, 5), (2, 6), (3, 7),
          (2, 4), (3, 5),
          (1, 2), (3, 4), (5, 6)]
_BITONIC8 = [(0, 4), (1, 5), (2, 6), (3, 7),
             (0, 2), (1, 3), (4, 6), (5, 7),
             (0, 1), (2, 3), (4, 5), (6, 7)]


def _sc_body(x_hbm, out_hbm, xbuf, cntbuf, thrbuf, sem):
    c = lax.axis_index("c")
    s = lax.axis_index("s")
    wid = s * _NC + c
    row0 = wid * _RPW

    pltpu.async_copy(x_hbm.at[pl.ds(row0, _RPW)], xbuf, sem).wait()

    iota = lax.broadcasted_iota(jnp.int32, (_L,), 0)
    neginf = jnp.full((_L,), -jnp.inf, jnp.float32)
    zeros = jnp.zeros((_L,), jnp.float32)

    def superblock(sb, cacc):
        ridx = [sb * _SB + g * _L + iota for g in range(_G)]

        r = [neginf] * (_G * _TOP_K)
        for it in range(_NUM_EXPERTS // _TOP_K):
            eb = it * _TOP_K
            for g in range(_G):
                b = [plsc.load_gather(
                        xbuf, [ridx[g], jnp.full((_L,), eb + k, jnp.int32)])
                     for k in range(_TOP_K)]
                for i, j in _SORT8:
                    lo = jnp.minimum(b[i], b[j])
                    b[j] = jnp.maximum(b[i], b[j])
                    b[i] = lo
                m = [jnp.maximum(r[g * _TOP_K + i], b[i])
                     for i in range(_TOP_K)]
                for i, j in _BITONIC8:
                    hi = jnp.maximum(m[i], m[j])
                    m[j] = jnp.minimum(m[i], m[j])
                    m[i] = hi
                for i in range(_TOP_K):
                    r[g * _TOP_K + i] = m[i]
        thr = [r[g * _TOP_K + _TOP_K - 1] for g in range(_G)]

        for g in range(_G):
            thrbuf[pl.ds(g * _L, _L)] = thr[g]
        cacc = list(cacc)
        for g in range(_G):
            for rr in range(_L):
                t16 = plsc.load_gather(
                    thrbuf, [jnp.full((_L,), g * _L + rr, jnp.int32)])
                row = sb * _SB + g * _L + rr
                rowv = jnp.full((_L,), row, jnp.int32)
                for j in range(_NUM_EXPERTS // _L):
                    v = plsc.load_gather(xbuf, [rowv, j * _L + iota])
                    cacc[j] = cacc[j] + jnp.where(v >= t16, 1.0, 0.0)
        return tuple(cacc)

    cacc = lax.fori_loop(0, _NSB, superblock, tuple([zeros] * 4))
    for j in range(4):
        cntbuf[pl.ds(j * _L, _L)] = cacc[j]
    pltpu.sync_copy(cntbuf, out_hbm.at[wid])


@functools.partial(
    pl.kernel,
    out_type=jax.ShapeDtypeStruct((_NW, _NUM_EXPERTS), jnp.float32),
    mesh=plsc.VectorSubcoreMesh(core_axis_name="c", subcore_axis_name="s"),
    scratch_types=[
        pltpu.VMEM((_RPW, _NUM_EXPERTS), jnp.float32),
        pltpu.VMEM((_NUM_EXPERTS,), jnp.float32),
        pltpu.VMEM((_SB,), jnp.float32),
        pltpu.SemaphoreType.DMA,
    ],
    compiler_params=pltpu.CompilerParams(needs_layout_passes=False,
                                         use_tc_tiling_on_sc=False),
)
def _sc_counts(x_hbm, out_hbm, xbuf, cntbuf, thrbuf, sem):
    _sc_body(x_hbm, out_hbm, xbuf, cntbuf, thrbuf, sem)


def _tc_body(x_ref, acc_ref):
    pi = pl.program_id(0)

    @pl.when(pi == 0)
    def _init():
        acc_ref[...] = jnp.zeros_like(acc_ref)

    x = x_ref[...]
    m = jnp.max(x, axis=1, keepdims=True)
    ex = jnp.exp(x - m)
    s = jnp.sum(ex, axis=1, keepdims=True)
    lse = m + jnp.log(s)
    z_part = jnp.sum(lse * lse, keepdims=True)
    probs = ex / s
    prob_part = jnp.sum(probs, axis=0, keepdims=True)

    acc_ref[0:1, 0:_NUM_EXPERTS] += prob_part
    acc_ref[1:2, 0:1] += z_part


def _tc_softmax_stats(router_logits):
    return pl.pallas_call(
        _tc_body,
        grid=(_ROWS // _BLOCK,),
        in_specs=[pl.BlockSpec((_BLOCK, _NUM_EXPERTS), lambda i: (i, 0))],
        out_specs=pl.BlockSpec((8, 128), lambda i: (0, 0)),
        out_shape=jax.ShapeDtypeStruct((8, 128), jnp.float32),
    )(router_logits)


@jax.jit
def kernel(router_logits):
    cnt_parts = _sc_counts(router_logits)
    acc = _tc_softmax_stats(router_logits)
    inv_n = 1.0 / _ROWS
    expert_prob = acc[0, :_NUM_EXPERTS] * inv_n
    expert_freq = jnp.sum(cnt_parts, axis=0) * inv_n
    z_loss = acc[1, 0] * inv_n
    global_loss = _NUM_EXPERTS * jnp.sum(expert_prob * expert_freq)
    return _ALPHA * global_loss + _GAMMA * z_loss


# speedup vs baseline: 1.9114x; 1.2150x over previous
import functools

import jax
import jax.numpy as jnp
from jax import lax
from jax.experimental import pallas as pl
from jax.experimental.pallas import tpu as pltpu
from jax.experimental.pallas import tpu_sc as plsc

_NUM_EXPERTS = 64
_TOP_K = 8
_ALPHA = 0.01
_GAMMA = 0.001
_ROWS = 32768
_LANES = 128
_EH = _NUM_EXPERTS // 8
_RH = _ROWS // _LANES

_NC = 2
_NS = 16
_NW = _NC * _NS
_L = 16
_RPW = _ROWS // _NW
_RHW = _RPW // _LANES
_G = 2
_SB = _G * _L
_NSB = _RPW // _SB

_SORT8 = [(0, 1), (2, 3), (4, 5), (6, 7),
          (0, 2), (1, 3), (4, 6), (5, 7),
          (1, 2), (5, 6),
          (0, 4), (1, 5), (2, 6), (3, 7),
          (2, 4), (3, 5),
          (1, 2), (3, 4), (5, 6)]
_BITONIC8 = [(0, 4), (1, 5), (2, 6), (3, 7),
             (0, 2), (1, 3), (4, 6), (5, 7),
             (0, 1), (2, 3), (4, 5), (6, 7)]


def _sc_body(x_hbm, out_hbm, xbuf, cntbuf, thrbuf, sem):
    c = lax.axis_index("c")
    s = lax.axis_index("s")
    wid = s * _NC + c
    rhi0 = wid * _RHW

    pltpu.async_copy(x_hbm.at[:, pl.ds(rhi0, _RHW)], xbuf, sem).wait()

    iota = lax.broadcasted_iota(jnp.int32, (_L,), 0)
    neginf = jnp.full((_L,), -jnp.inf, jnp.float32)
    zeros = jnp.zeros((_L,), jnp.float32)

    def superblock(sb, cacc):
        rhi = sb // (_LANES // _SB)
        rlo0 = (sb % (_LANES // _SB)) * _SB

        r = [neginf] * (_G * _TOP_K)
        for it in range(_TOP_K):
            eb = it * _TOP_K
            for g in range(_G):
                b = [xbuf[(eb + k) // 8, rhi, (eb + k) % 8,
                          pl.ds(rlo0 + g * _L, _L)]
                     for k in range(_TOP_K)]
                for i, j in _SORT8:
                    lo = jnp.minimum(b[i], b[j])
                    b[j] = jnp.maximum(b[i], b[j])
                    b[i] = lo
                m = [jnp.maximum(r[g * _TOP_K + i], b[i])
                     for i in range(_TOP_K)]
                for i, j in _BITONIC8:
                    hi = jnp.maximum(m[i], m[j])
                    m[j] = jnp.minimum(m[i], m[j])
                    m[i] = hi
                for i in range(_TOP_K):
                    r[g * _TOP_K + i] = m[i]
        thr = [r[g * _TOP_K + _TOP_K - 1] for g in range(_G)]

        for g in range(_G):
            thrbuf[pl.ds(g * _L, _L)] = thr[g]
        cacc = list(cacc)
        ehi = [2 * j + iota // 8 for j in range(_NUM_EXPERTS // _L)]
        elo = iota % 8
        rhiv = jnp.zeros((_L,), jnp.int32) + rhi
        for g in range(_G):
            for rr in range(_L):
                t16 = plsc.load_gather(
                    thrbuf, [jnp.full((_L,), g * _L + rr, jnp.int32)])
                rlov = jnp.zeros((_L,), jnp.int32) + (rlo0 + g * _L + rr)
                for j in range(_NUM_EXPERTS // _L):
                    v = plsc.load_gather(xbuf, [ehi[j], rhiv, elo, rlov])
                    cacc[j] = cacc[j] + jnp.where(v >= t16, 1.0, 0.0)
        return tuple(cacc)

    cacc = lax.fori_loop(0, _NSB, superblock, tuple([zeros] * 4))
    for j in range(4):
        cntbuf[pl.ds(j * _L, _L)] = cacc[j]
    pltpu.sync_copy(cntbuf, out_hbm.at[wid])


@functools.partial(
    pl.kernel,
    out_type=jax.ShapeDtypeStruct((_NW, _NUM_EXPERTS), jnp.float32),
    mesh=plsc.VectorSubcoreMesh(core_axis_name="c", subcore_axis_name="s"),
    scratch_types=[
        pltpu.VMEM((_EH, _RHW, 8, _LANES), jnp.float32),
        pltpu.VMEM((_NUM_EXPERTS,), jnp.float32),
        pltpu.VMEM((_SB,), jnp.float32),
        pltpu.SemaphoreType.DMA,
    ],
    compiler_params=pltpu.CompilerParams(needs_layout_passes=False,
                                         use_tc_tiling_on_sc=False),
)
def _sc_counts(x_hbm, out_hbm, xbuf, cntbuf, thrbuf, sem):
    _sc_body(x_hbm, out_hbm, xbuf, cntbuf, thrbuf, sem)


_TCB = 4096


def _tc_body(xt_ref, acc_ref):
    pi = pl.program_id(0)

    @pl.when(pi == 0)
    def _init():
        acc_ref[...] = jnp.zeros_like(acc_ref)

    x = xt_ref[...]
    m = jnp.max(x, axis=0, keepdims=True)
    ex = jnp.exp(x - m)
    s = jnp.sum(ex, axis=0, keepdims=True)
    lse = m + jnp.log(s)
    z_part = jnp.sum(lse * lse, axis=1, keepdims=True)
    prob_part = jnp.sum(ex / s, axis=1, keepdims=True)

    acc_ref[0:_NUM_EXPERTS, 0:1] += prob_part
    acc_ref[0:1, 1:2] += z_part


def _tc_softmax_stats(xt):
    return pl.pallas_call(
        _tc_body,
        grid=(_ROWS // _TCB,),
        in_specs=[pl.BlockSpec((_NUM_EXPERTS, _TCB), lambda i: (0, i))],
        out_specs=pl.BlockSpec((_NUM_EXPERTS, 128), lambda i: (0, 0)),
        out_shape=jax.ShapeDtypeStruct((_NUM_EXPERTS, 128), jnp.float32),
    )(xt)


@jax.jit
def kernel(router_logits):
    xt = router_logits.T
    xv = xt.reshape(_EH, 8, _RH, _LANES).transpose(0, 2, 1, 3)
    cnt_parts = _sc_counts(xv)
    acc = _tc_softmax_stats(xt)
    inv_n = 1.0 / _ROWS
    expert_prob = acc[:, 0] * inv_n
    expert_freq = jnp.sum(cnt_parts, axis=0) * inv_n
    z_loss = acc[0, 1] * inv_n
    global_loss = _NUM_EXPERTS * jnp.sum(expert_prob * expert_freq)
    return _ALPHA * global_loss + _GAMMA * z_loss


# speedup vs baseline: 1.9127x; 1.0007x over previous
import functools

import jax
import jax.numpy as jnp
from jax import lax
from jax.experimental import pallas as pl
from jax.experimental.pallas import tpu as pltpu
from jax.experimental.pallas import tpu_sc as plsc

_NUM_EXPERTS = 64
_TOP_K = 8
_ALPHA = 0.01
_GAMMA = 0.001
_ROWS = 32768
_LANES = 128
_EH = _NUM_EXPERTS // 8
_RH = _ROWS // _LANES

_NC = 2
_NS = 16
_NW = _NC * _NS
_L = 16
_RPW = _ROWS // _NW
_RHW = _RPW // _LANES
_G = 2
_SB = _G * _L
_NSB = _RPW // _SB

_SORT8 = [(0, 1), (2, 3), (4, 5), (6, 7),
          (0, 2), (1, 3), (4, 6), (5, 7),
          (1, 2), (5, 6),
          (0, 4), (1, 5), (2, 6), (3, 7),
          (2, 4), (3, 5),
          (1, 2), (3, 4), (5, 6)]
_BITONIC8 = [(0, 4), (1, 5), (2, 6), (3, 7),
             (0, 2), (1, 3), (4, 6), (5, 7),
             (0, 1), (2, 3), (4, 5), (6, 7)]


def _sc_body(x_hbm, out_hbm, xbuf, cntbuf, thrbuf, sem):
    c = lax.axis_index("c")
    s = lax.axis_index("s")
    wid = s * _NC + c
    rhi0 = wid * _RHW

    pltpu.async_copy(x_hbm.at[:, pl.ds(rhi0, _RHW)], xbuf, sem).wait()

    iota = lax.broadcasted_iota(jnp.int32, (_L,), 0)
    neginf = jnp.full((_L,), -jnp.inf, jnp.float32)
    zeros = jnp.zeros((_L,), jnp.float32)

    def superblock(sb, cacc):
        rhi = sb // (_LANES // _SB)
        rlo0 = (sb % (_LANES // _SB)) * _SB

        r = [neginf] * (_G * _TOP_K)
        for it in range(_TOP_K):
            eb = it * _TOP_K
            for g in range(_G):
                b = [xbuf[(eb + k) // 8, rhi, (eb + k) % 8,
                          pl.ds(rlo0 + g * _L, _L)]
                     for k in range(_TOP_K)]
                for i, j in _SORT8:
                    lo = jnp.minimum(b[i], b[j])
                    b[j] = jnp.maximum(b[i], b[j])
                    b[i] = lo
                m = [jnp.maximum(r[g * _TOP_K + i], b[i])
                     for i in range(_TOP_K)]
                for i, j in _BITONIC8:
                    hi = jnp.maximum(m[i], m[j])
                    m[j] = jnp.minimum(m[i], m[j])
                    m[i] = hi
                for i in range(_TOP_K):
                    r[g * _TOP_K + i] = m[i]
        thr = [r[g * _TOP_K + _TOP_K - 1] for g in range(_G)]

        for g in range(_G):
            thrbuf[pl.ds(g * _L, _L)] = thr[g]
        cacc = list(cacc)
        ehi = [2 * j + iota // 8 for j in range(_NUM_EXPERTS // _L)]
        elo = iota % 8
        rhiv = jnp.zeros((_L,), jnp.int32) + rhi
        for g in range(_G):
            for rr in range(_L):
                t16 = plsc.load_gather(
                    thrbuf, [jnp.full((_L,), g * _L + rr, jnp.int32)])
                rlov = jnp.zeros((_L,), jnp.int32) + (rlo0 + g * _L + rr)
                for j in range(_NUM_EXPERTS // _L):
                    v = plsc.load_gather(xbuf, [ehi[j], rhiv, elo, rlov])
                    cacc[j] = cacc[j] + jnp.where(v >= t16, 1.0, 0.0)
        return tuple(cacc)

    cacc = lax.fori_loop(0, _NSB, superblock, tuple([zeros] * 4))
    for j in range(4):
        cntbuf[pl.ds(j * _L, _L)] = cacc[j]
    pltpu.sync_copy(cntbuf, out_hbm.at[wid])


@functools.partial(
    pl.kernel,
    out_type=jax.ShapeDtypeStruct((_NW, _NUM_EXPERTS), jnp.float32),
    mesh=plsc.VectorSubcoreMesh(core_axis_name="c", subcore_axis_name="s"),
    scratch_types=[
        pltpu.VMEM((_EH, _RHW, 8, _LANES), jnp.float32),
        pltpu.VMEM((_NUM_EXPERTS,), jnp.float32),
        pltpu.VMEM((_SB,), jnp.float32),
        pltpu.SemaphoreType.DMA,
    ],
    compiler_params=pltpu.CompilerParams(needs_layout_passes=False,
                                         use_tc_tiling_on_sc=False),
)
def _sc_counts(x_hbm, out_hbm, xbuf, cntbuf, thrbuf, sem):
    _sc_body(x_hbm, out_hbm, xbuf, cntbuf, thrbuf, sem)


_TCB = 4096


def _tc_body(xt_hbm, acc_ref, buf, sem):
    pi = pl.program_id(0)
    nb = pl.num_programs(0)

    @pl.when(pi == 0)
    def _init():
        acc_ref[...] = jnp.zeros_like(acc_ref)
        pltpu.make_async_copy(
            xt_hbm.at[:, pl.ds(0, _TCB)], buf.at[0], sem.at[0]).start()

    @pl.when(pi + 1 < nb)
    def _prefetch():
        pltpu.make_async_copy(
            xt_hbm.at[:, pl.ds((pi + 1) * _TCB, _TCB)],
            buf.at[(pi + 1) % 2], sem.at[(pi + 1) % 2]).start()

    pltpu.make_async_copy(
        xt_hbm.at[:, pl.ds(pi * _TCB, _TCB)], buf.at[pi % 2], sem.at[pi % 2]
    ).wait()

    x = buf[pi % 2]
    m = jnp.max(x, axis=0, keepdims=True)
    ex = jnp.exp(x - m)
    s = jnp.sum(ex, axis=0, keepdims=True)
    lse = m + jnp.log(s)
    z_part = jnp.sum(lse * lse, axis=1, keepdims=True)
    prob_part = jnp.sum(ex / s, axis=1, keepdims=True)

    acc_ref[0:_NUM_EXPERTS, 0:1] += prob_part
    acc_ref[0:1, 1:2] += z_part


def _tc_softmax_stats(xt):
    xt = pltpu.with_memory_space_constraint(xt, pltpu.MemorySpace.HBM)
    return pl.pallas_call(
        _tc_body,
        grid=(_ROWS // _TCB,),
        in_specs=[pl.BlockSpec(memory_space=pl.ANY)],
        out_specs=pl.BlockSpec((_NUM_EXPERTS, 128), lambda i: (0, 0)),
        out_shape=jax.ShapeDtypeStruct((_NUM_EXPERTS, 128), jnp.float32),
        scratch_shapes=[
            pltpu.VMEM((2, _NUM_EXPERTS, _TCB), jnp.float32),
            pltpu.SemaphoreType.DMA((2,)),
        ],
    )(xt)


@jax.jit
def kernel(router_logits):
    xt = router_logits.T
    xv = xt.reshape(_EH, 8, _RH, _LANES).transpose(0, 2, 1, 3)
    cnt_parts = _sc_counts(xv)
    acc = _tc_softmax_stats(xt)
    inv_n = 1.0 / _ROWS
    expert_prob = acc[:, 0] * inv_n
    expert_freq = jnp.sum(cnt_parts, axis=0) * inv_n
    z_loss = acc[0, 1] * inv_n
    global_loss = _NUM_EXPERTS * jnp.sum(expert_prob * expert_freq)
    return _ALPHA * global_loss + _GAMMA * z_loss


# speedup vs baseline: 3.5680x; 1.8655x over previous
import functools

import jax
import jax.numpy as jnp
from jax import lax
from jax.experimental import pallas as pl
from jax.experimental.pallas import tpu as pltpu
from jax.experimental.pallas import tpu_sc as plsc

_NUM_EXPERTS = 64
_TOP_K = 8
_ALPHA = 0.01
_GAMMA = 0.001
_ROWS = 32768
_LANES = 128
_EH = _NUM_EXPERTS // 8
_RH = _ROWS // _LANES

_NC = 2
_NS = 16
_NW = _NC * _NS
_L = 16
_RPW = _ROWS // _NW
_RHW = _RPW // _LANES
_G = 2
_SB = _G * _L
_NSB = _RPW // _SB

_SORT8 = [(0, 1), (2, 3), (4, 5), (6, 7),
          (0, 2), (1, 3), (4, 6), (5, 7),
          (1, 2), (5, 6),
          (0, 4), (1, 5), (2, 6), (3, 7),
          (2, 4), (3, 5),
          (1, 2), (3, 4), (5, 6)]
_BITONIC8 = [(0, 4), (1, 5), (2, 6), (3, 7),
             (0, 2), (1, 3), (4, 6), (5, 7),
             (0, 1), (2, 3), (4, 5), (6, 7)]


def _sc_body(x_hbm, out_hbm, xbuf, cntbuf, thrbuf, sem):
    c = lax.axis_index("c")
    s = lax.axis_index("s")
    wid = s * _NC + c
    rhi0 = wid * _RHW

    pltpu.async_copy(x_hbm.at[:, pl.ds(rhi0, _RHW)], xbuf, sem).wait()

    iota = lax.broadcasted_iota(jnp.int32, (_L,), 0)
    neginf = jnp.full((_L,), -jnp.inf, jnp.float32)
    zeros = jnp.zeros((_L,), jnp.float32)

    def superblock(sb, cacc):
        rhi = sb // (_LANES // _SB)
        rlo0 = (sb % (_LANES // _SB)) * _SB

        r = [neginf] * (_G * _TOP_K)
        for it in range(_TOP_K):
            eb = it * _TOP_K
            for g in range(_G):
                b = [xbuf[(eb + k) // 8, rhi, (eb + k) % 8,
                          pl.ds(rlo0 + g * _L, _L)]
                     for k in range(_TOP_K)]
                for i, j in _SORT8:
                    lo = jnp.minimum(b[i], b[j])
                    b[j] = jnp.maximum(b[i], b[j])
                    b[i] = lo
                m = [jnp.maximum(r[g * _TOP_K + i], b[i])
                     for i in range(_TOP_K)]
                for i, j in _BITONIC8:
                    hi = jnp.maximum(m[i], m[j])
                    m[j] = jnp.minimum(m[i], m[j])
                    m[i] = hi
                for i in range(_TOP_K):
                    r[g * _TOP_K + i] = m[i]
        thr = [r[g * _TOP_K + _TOP_K - 1] for g in range(_G)]

        cacc = list(cacc)
        for e in range(_NUM_EXPERTS):
            pc = None
            for g in range(_G):
                v = xbuf[e // 8, rhi, e % 8, pl.ds(rlo0 + g * _L, _L)]
                p = plsc.all_reduce_population_count(v >= thr[g])
                pc = p if pc is None else pc + p
            onehot = (iota == (e % _L)).astype(jnp.float32)
            cacc[e // _L] = cacc[e // _L] + onehot * pc.astype(jnp.float32)
        return tuple(cacc)

    cacc = lax.fori_loop(0, _NSB, superblock, tuple([zeros] * 4))
    for j in range(4):
        cntbuf[pl.ds(j * _L, _L)] = cacc[j]
    pltpu.sync_copy(cntbuf, out_hbm.at[wid])


@functools.partial(
    pl.kernel,
    out_type=jax.ShapeDtypeStruct((_NW, _NUM_EXPERTS), jnp.float32),
    mesh=plsc.VectorSubcoreMesh(core_axis_name="c", subcore_axis_name="s"),
    scratch_types=[
        pltpu.VMEM((_EH, _RHW, 8, _LANES), jnp.float32),
        pltpu.VMEM((_NUM_EXPERTS,), jnp.float32),
        pltpu.VMEM((_SB,), jnp.float32),
        pltpu.SemaphoreType.DMA,
    ],
    compiler_params=pltpu.CompilerParams(needs_layout_passes=False,
                                         use_tc_tiling_on_sc=False),
)
def _sc_counts(x_hbm, out_hbm, xbuf, cntbuf, thrbuf, sem):
    _sc_body(x_hbm, out_hbm, xbuf, cntbuf, thrbuf, sem)


_TCB = 4096


def _tc_body(xt_hbm, acc_ref, buf, sem):
    pi = pl.program_id(0)
    nb = pl.num_programs(0)

    @pl.when(pi == 0)
    def _init():
        acc_ref[...] = jnp.zeros_like(acc_ref)
        pltpu.make_async_copy(
            xt_hbm.at[:, pl.ds(0, _TCB)], buf.at[0], sem.at[0]).start()

    @pl.when(pi + 1 < nb)
    def _prefetch():
        pltpu.make_async_copy(
            xt_hbm.at[:, pl.ds((pi + 1) * _TCB, _TCB)],
            buf.at[(pi + 1) % 2], sem.at[(pi + 1) % 2]).start()

    pltpu.make_async_copy(
        xt_hbm.at[:, pl.ds(pi * _TCB, _TCB)], buf.at[pi % 2], sem.at[pi % 2]
    ).wait()

    x = buf[pi % 2]
    m = jnp.max(x, axis=0, keepdims=True)
    ex = jnp.exp(x - m)
    s = jnp.sum(ex, axis=0, keepdims=True)
    lse = m + jnp.log(s)
    z_part = jnp.sum(lse * lse, axis=1, keepdims=True)
    prob_part = jnp.sum(ex / s, axis=1, keepdims=True)

    acc_ref[0:_NUM_EXPERTS, 0:1] += prob_part
    acc_ref[0:1, 1:2] += z_part


def _tc_softmax_stats(xt):
    xt = pltpu.with_memory_space_constraint(xt, pltpu.MemorySpace.HBM)
    return pl.pallas_call(
        _tc_body,
        grid=(_ROWS // _TCB,),
        in_specs=[pl.BlockSpec(memory_space=pl.ANY)],
        out_specs=pl.BlockSpec((_NUM_EXPERTS, 128), lambda i: (0, 0)),
        out_shape=jax.ShapeDtypeStruct((_NUM_EXPERTS, 128), jnp.float32),
        scratch_shapes=[
            pltpu.VMEM((2, _NUM_EXPERTS, _TCB), jnp.float32),
            pltpu.SemaphoreType.DMA((2,)),
        ],
    )(xt)


@jax.jit
def kernel(router_logits):
    xt = router_logits.T
    xv = xt.reshape(_EH, 8, _RH, _LANES).transpose(0, 2, 1, 3)
    cnt_parts = _sc_counts(xv)
    acc = _tc_softmax_stats(xt)
    inv_n = 1.0 / _ROWS
    expert_prob = acc[:, 0] * inv_n
    expert_freq = jnp.sum(cnt_parts, axis=0) * inv_n
    z_loss = acc[0, 1] * inv_n
    global_loss = _NUM_EXPERTS * jnp.sum(expert_prob * expert_freq)
    return _ALPHA * global_loss + _GAMMA * z_loss
